# baseline (speedup 1.0000x reference)
import functools

import jax
import jax.numpy as jnp
from jax import lax
from jax.experimental import pallas as pl
from jax.experimental.pallas import tpu as pltpu
from jax.experimental.pallas import tpu_sc as plsc

L = 16
NT = 16


def _lane_total(v):
    return plsc.cummax(jnp.flip(plsc.cumsum(v), 0))


def _make_sc_call(chunk, per_tile, B, M, D, DS):
    n_vec = chunk // L
    groups = per_tile // L
    mesh = plsc.VectorSubcoreMesh(core_axis_name="c", subcore_axis_name="s",
                                  num_cores=1)

    @functools.partial(
        pl.kernel,
        out_type=jax.ShapeDtypeStruct((L,), jnp.float32),
        mesh=mesh,
        compiler_params=pltpu.CompilerParams(needs_layout_passes=False,
                                             use_tc_tiling_on_sc=False),
        scratch_types=[
            pltpu.VMEM((chunk,), jnp.int32),
            pltpu.VMEM((B, L), jnp.int32),
            pltpu.VMEM_SHARED((NT, B, L), jnp.int32),
            pltpu.VMEM((NT, B, L), jnp.int32),
            pltpu.VMEM((per_tile,), jnp.int32),
            pltpu.VMEM((per_tile,), jnp.int32),
            pltpu.VMEM((D, per_tile), jnp.int32),
            pltpu.VMEM((D, per_tile), jnp.float32),
            pltpu.VMEM((D, per_tile), jnp.float32),
            pltpu.VMEM((D + 1, L), jnp.float32),
            pltpu.VMEM_SHARED((NT, D + 1, L), jnp.float32),
            pltpu.VMEM((NT, D + 1, L), jnp.float32),
            pltpu.VMEM((L,), jnp.float32),
            pltpu.SemaphoreType.DMA,
        ],
    )
    def sc_kernel(flat_hbm, bi_hbm, ind_hbm, mask_hbm, tgtT_hbm, out_hbm,
                  bi_chunk, cnt_stage, cnt_sh, cnt_all, ind_v, mask_v,
                  idxd_v, cols_v, tgt_v, part_stage, part_sh, part_all,
                  out_stage, sem):
        sid = lax.axis_index("s")
        lane = jnp.arange(L, dtype=jnp.int32)
        zero_i = jnp.zeros((L,), jnp.int32)
        zero_f = jnp.zeros((L,), jnp.float32)
        one_i = jnp.ones((L,), jnp.int32)
        one_f = jnp.ones((L,), jnp.float32)

        pltpu.sync_copy(bi_hbm.at[pl.ds(sid * chunk, chunk)], bi_chunk)

        def scan_body(i, accs):
            v = bi_chunk[pl.ds(i * L, L)]
            return tuple(accs[k] + jnp.where(v < (k + 1), one_i, zero_i)
                         for k in range(B))

        accs = lax.fori_loop(0, n_vec, scan_body, (zero_i,) * B)
        for k in range(B):
            cnt_stage[k, :] = accs[k]
        pltpu.sync_copy(cnt_stage, cnt_sh.at[sid])
        plsc.subcore_barrier()
        pltpu.sync_copy(cnt_sh, cnt_all)

        totals = []
        for k in range(B):
            t = zero_i
            for w in range(NT):
                t = t + cnt_all[w, k, :]
            totals.append(_lane_total(t))

        def start_count_for(b):
            st_g = zero_i
            ct_g = zero_i
            prev = zero_i
            for k in range(B):
                sel = b == k
                st_g = jnp.where(sel, prev, st_g)
                ct_g = jnp.where(sel, totals[k] - prev, ct_g)
                prev = totals[k]
            return st_g, ct_g

        pbase = sid * per_tile
        pltpu.sync_copy(ind_hbm.at[pl.ds(pbase, per_tile)], ind_v)
        pltpu.sync_copy(mask_hbm.at[pl.ds(pbase, per_tile)], mask_v)
        pltpu.sync_copy(tgtT_hbm.at[:, pl.ds(pbase, per_tile)], tgt_v)

        numv = zero_f
        for g in range(groups):
            p = pbase + g * L + lane
            b = p // M
            st_g, ct_g = start_count_for(b)
            ind_g = ind_v[pl.ds(g * L, L)]
            clipped = jnp.clip(ind_g, 0, jnp.maximum(ct_g - one_i, zero_i))
            base_e = (st_g + clipped) * DS
            for d in range(D):
                idxd_v[d, pl.ds(g * L, L)] = base_e + d
            numv = numv + mask_v[pl.ds(g * L, L)].astype(jnp.float32)

        copies = [pltpu.async_copy(flat_hbm.at[idxd_v.at[d]], cols_v.at[d],
                                   sem) for d in range(D)]
        for c in copies:
            c.wait()

        acc = [zero_f for _ in range(D)]
        for g in range(groups):
            p = pbase + g * L + lane
            b = p // M
            _, ct_g = start_count_for(b)
            cpos = ct_g > 0
            m = mask_v[pl.ds(g * L, L)].astype(jnp.float32)
            for d in range(D):
                pred = cols_v[d, pl.ds(g * L, L)]
                tgt = tgt_v[d, pl.ds(g * L, L)]
                pred = jnp.where(cpos, pred, zero_f)
                tn = tgt != tgt
                pn = pred != pred
                vm = m * jnp.where(tn, zero_f, one_f) \
                       * jnp.where(pn, zero_f, one_f)
                ps = jnp.where(pn, zero_f, pred)
                ts = jnp.where(tn, zero_f, tgt)
                acc[d] = acc[d] + jnp.abs(ps * vm - ts * vm)

        for d in range(D):
            part_stage[d, :] = acc[d]
        part_stage[D, :] = numv
        pltpu.sync_copy(part_stage, part_sh.at[sid])
        plsc.subcore_barrier()

        @pl.when(sid == 0)
        def _():
            pltpu.sync_copy(part_sh, part_all)
            loss = zero_f
            nv = zero_f
            for w in range(NT):
                nv = nv + part_all[w, D, :]
            for d in range(D):
                t = zero_f
                for w in range(NT):
                    t = t + part_all[w, d, :]
                loss = jnp.where(lane == d, _lane_total(t), loss)
            num = _lane_total(nv)
            loss = loss / jnp.maximum(num, one_f)
            loss = jnp.where(loss != loss, zero_f, loss)
            loss = jnp.where(num == 0.0, zero_f, loss)
            loss = jnp.where(lane < D, loss, zero_f)
            out_stage[...] = loss
            pltpu.sync_copy(out_stage, out_hbm)

    return sc_kernel


def kernel(output, mask, ind, target, batch_index):
    V, D = output.shape
    B, M = mask.shape
    N = batch_index.shape[0]
    P = B * M

    chunk = ((N + NT - 1) // NT + L - 1) // L * L
    N_pad = chunk * NT
    per_tile = ((P + NT - 1) // NT + L - 1) // L * L
    PP = per_tile * NT

    bi = jnp.pad(batch_index.astype(jnp.int32), (0, N_pad - N),
                 constant_values=B)
    ind_flat = jnp.pad(ind.reshape(-1).astype(jnp.int32), (0, PP - P))
    mask_flat = jnp.pad(mask.reshape(-1).astype(jnp.int32), (0, PP - P))
    tgtT = jnp.pad(target.reshape(P, D).astype(jnp.float32),
                   ((0, PP - P), (0, 0))).T

    DS = 16
    flat = jnp.pad(output.astype(jnp.float32),
                   ((0, 0), (0, DS - D))).reshape(-1)
    sc = _make_sc_call(chunk, per_tile, B, M, D, DS)
    out16 = sc(flat, bi, ind_flat, mask_flat, tgtT)
    return out16[:D]


# speedup vs baseline: 1.0069x; 1.0069x over previous
import functools

import jax
import jax.numpy as jnp
from jax import lax
from jax.experimental import pallas as pl
from jax.experimental.pallas import tpu as pltpu
from jax.experimental.pallas import tpu_sc as plsc

L = 16
NT = 16


def _lane_total(v):
    return plsc.cummax(jnp.flip(plsc.cumsum(v), 0))


def _make_sc_call(chunk, per_tile, B, M, D):
    n_vec = chunk // L
    groups = per_tile // L
    mesh = plsc.VectorSubcoreMesh(core_axis_name="c", subcore_axis_name="s",
                                  num_cores=1)

    @functools.partial(
        pl.kernel,
        out_type=jax.ShapeDtypeStruct((L,), jnp.float32),
        mesh=mesh,
        compiler_params=pltpu.CompilerParams(needs_layout_passes=False,
                                             use_tc_tiling_on_sc=False),
        scratch_types=[
            pltpu.VMEM((chunk,), jnp.int32),
            pltpu.VMEM((B, L), jnp.int32),
            pltpu.VMEM_SHARED((NT, B, L), jnp.int32),
            pltpu.VMEM((NT, B, L), jnp.int32),
            pltpu.VMEM((per_tile,), jnp.int32),
            pltpu.VMEM((per_tile,), jnp.int32),
            pltpu.VMEM((per_tile,), jnp.int32),
            pltpu.VMEM((per_tile, 16), jnp.float32),
            pltpu.VMEM((D, per_tile), jnp.float32),
            pltpu.VMEM((D + 1, L), jnp.float32),
            pltpu.VMEM_SHARED((NT, D + 1, L), jnp.float32),
            pltpu.VMEM((NT, D + 1, L), jnp.float32),
            pltpu.VMEM((L,), jnp.float32),
            pltpu.SemaphoreType.DMA,
        ],
    )
    def sc_kernel(flat_hbm, bi_hbm, ind_hbm, mask_hbm, tgtT_hbm, out_hbm,
                  bi_chunk, cnt_stage, cnt_sh, cnt_all, ind_v, mask_v,
                  idx_v, rows_v, tgt_v, part_stage, part_sh, part_all,
                  out_stage, sem):
        sid = lax.axis_index("s")
        lane = jnp.arange(L, dtype=jnp.int32)
        zero_i = jnp.zeros((L,), jnp.int32)
        zero_f = jnp.zeros((L,), jnp.float32)
        one_i = jnp.ones((L,), jnp.int32)
        one_f = jnp.ones((L,), jnp.float32)

        pltpu.sync_copy(bi_hbm.at[pl.ds(sid * chunk, chunk)], bi_chunk)

        def scan_body(i, accs):
            v = bi_chunk[pl.ds(i * L, L)]
            return tuple(accs[k] + jnp.where(v < (k + 1), one_i, zero_i)
                         for k in range(B))

        accs = lax.fori_loop(0, n_vec, scan_body, (zero_i,) * B)
        for k in range(B):
            cnt_stage[k, :] = accs[k]
        pltpu.sync_copy(cnt_stage, cnt_sh.at[sid])
        plsc.subcore_barrier()
        pltpu.sync_copy(cnt_sh, cnt_all)

        totals = []
        for k in range(B):
            t = zero_i
            for w in range(NT):
                t = t + cnt_all[w, k, :]
            totals.append(_lane_total(t))

        def start_count_for(b):
            st_g = zero_i
            ct_g = zero_i
            prev = zero_i
            for k in range(B):
                sel = b == k
                st_g = jnp.where(sel, prev, st_g)
                ct_g = jnp.where(sel, totals[k] - prev, ct_g)
                prev = totals[k]
            return st_g, ct_g

        pbase = sid * per_tile
        pltpu.sync_copy(ind_hbm.at[pl.ds(pbase, per_tile)], ind_v)
        pltpu.sync_copy(mask_hbm.at[pl.ds(pbase, per_tile)], mask_v)
        pltpu.sync_copy(tgtT_hbm.at[:, pl.ds(pbase, per_tile)], tgt_v)

        numv = zero_f
        for g in range(groups):
            p = pbase + g * L + lane
            b = p // M
            st_g, ct_g = start_count_for(b)
            ind_g = ind_v[pl.ds(g * L, L)]
            clipped = jnp.clip(ind_g, 0, jnp.maximum(ct_g - one_i, zero_i))
            idx_v[pl.ds(g * L, L)] = st_g + clipped
            numv = numv + mask_v[pl.ds(g * L, L)].astype(jnp.float32)

        pltpu.async_copy(flat_hbm.at[idx_v], rows_v, sem).wait()

        acc = [zero_f for _ in range(D)]
        for g in range(groups):
            p = pbase + g * L + lane
            b = p // M
            _, ct_g = start_count_for(b)
            cpos = ct_g > 0
            m = mask_v[pl.ds(g * L, L)].astype(jnp.float32)
            row_ids = jnp.full((L,), g * L, jnp.int32) + lane
            for d in range(D):
                pred = plsc.load_gather(
                    rows_v, [row_ids, jnp.full((L,), d, jnp.int32)])
                tgt = tgt_v[d, pl.ds(g * L, L)]
                pred = jnp.where(cpos, pred, zero_f)
                tn = tgt != tgt
                pn = pred != pred
                vm = m * jnp.where(tn, zero_f, one_f) \
                       * jnp.where(pn, zero_f, one_f)
                ps = jnp.where(pn, zero_f, pred)
                ts = jnp.where(tn, zero_f, tgt)
                acc[d] = acc[d] + jnp.abs(ps * vm - ts * vm)

        for d in range(D):
            part_stage[d, :] = acc[d]
        part_stage[D, :] = numv
        pltpu.sync_copy(part_stage, part_sh.at[sid])
        plsc.subcore_barrier()

        @pl.when(sid == 0)
        def _():
            pltpu.sync_copy(part_sh, part_all)
            loss = zero_f
            nv = zero_f
            for w in range(NT):
                nv = nv + part_all[w, D, :]
            for d in range(D):
                t = zero_f
                for w in range(NT):
                    t = t + part_all[w, d, :]
                loss = jnp.where(lane == d, _lane_total(t), loss)
            num = _lane_total(nv)
            loss = loss / jnp.maximum(num, one_f)
            loss = jnp.where(loss != loss, zero_f, loss)
            loss = jnp.where(num == 0.0, zero_f, loss)
            loss = jnp.where(lane < D, loss, zero_f)
            out_stage[...] = loss
            pltpu.sync_copy(out_stage, out_hbm)

    return sc_kernel


def kernel(output, mask, ind, target, batch_index):
    V, D = output.shape
    B, M = mask.shape
    N = batch_index.shape[0]
    P = B * M

    chunk = ((N + NT - 1) // NT + L - 1) // L * L
    N_pad = chunk * NT
    per_tile = ((P + NT - 1) // NT + L - 1) // L * L
    PP = per_tile * NT

    bi = jnp.pad(batch_index.astype(jnp.int32), (0, N_pad - N),
                 constant_values=B)
    ind_flat = jnp.pad(ind.reshape(-1).astype(jnp.int32), (0, PP - P))
    mask_flat = jnp.pad(mask.reshape(-1).astype(jnp.int32), (0, PP - P))
    tgtT = jnp.pad(target.reshape(P, D).astype(jnp.float32),
                   ((0, PP - P), (0, 0))).T

    sc = _make_sc_call(chunk, per_tile, B, M, D)
    out16 = sc(jnp.pad(output.astype(jnp.float32), ((0, 0), (0, 16 - D))),
               bi, ind_flat, mask_flat, tgtT)
    return out16[:D]


# speedup vs baseline: 1.2970x; 1.2881x over previous
import functools

import jax
import jax.numpy as jnp
from jax import lax
from jax.experimental import pallas as pl
from jax.experimental.pallas import tpu as pltpu
from jax.experimental.pallas import tpu_sc as plsc

L = 16
NT = 16


def _lane_total(v):
    return plsc.cummax(jnp.flip(plsc.cumsum(v), 0))


def _make_sc_call(chunk, per_tile, B, M, D, NBLK):
    n_vec = chunk // L
    groups = per_tile // L
    mesh = plsc.VectorSubcoreMesh(core_axis_name="c", subcore_axis_name="s",
                                  num_cores=1)

    @functools.partial(
        pl.kernel,
        out_type=jax.ShapeDtypeStruct((L,), jnp.float32),
        mesh=mesh,
        compiler_params=pltpu.CompilerParams(needs_layout_passes=False,
                                             use_tc_tiling_on_sc=False),
        scratch_types=[
            pltpu.VMEM((chunk,), jnp.int32),
            pltpu.VMEM((B, L), jnp.int32),
            pltpu.VMEM_SHARED((NT, B, L), jnp.int32),
            pltpu.VMEM((NT, B, L), jnp.int32),
            pltpu.VMEM((per_tile,), jnp.int32),
            pltpu.VMEM((per_tile,), jnp.int32),
            pltpu.VMEM((per_tile,), jnp.int32),
            pltpu.VMEM((per_tile,), jnp.int32),
            pltpu.VMEM((per_tile,), jnp.int32),
            pltpu.VMEM((per_tile, 128), jnp.float32),
            pltpu.VMEM((per_tile, 128), jnp.float32),
            pltpu.VMEM((D, per_tile), jnp.float32),
            pltpu.VMEM((D + 1, L), jnp.float32),
            pltpu.VMEM_SHARED((NT, D + 1, L), jnp.float32),
            pltpu.VMEM((NT, D + 1, L), jnp.float32),
            pltpu.VMEM((L,), jnp.float32),
            pltpu.SemaphoreType.DMA,
        ],
    )
    def sc_kernel(flat_hbm, bi_hbm, ind_hbm, mask_hbm, tgtT_hbm, out_hbm,
                  bi_chunk, cnt_stage, cnt_sh, cnt_all, ind_v, mask_v,
                  b0_v, b1_v, off_v, blk0_v, blk1_v, tgt_v, part_stage, part_sh, part_all,
                  out_stage, sem):
        sid = lax.axis_index("s")
        lane = jnp.arange(L, dtype=jnp.int32)
        nblk1 = jnp.full((L,), NBLK - 1, jnp.int32)
        nblk1 = jnp.full((L,), NBLK - 1, jnp.int32)
        zero_i = jnp.zeros((L,), jnp.int32)
        zero_f = jnp.zeros((L,), jnp.float32)
        one_i = jnp.ones((L,), jnp.int32)
        one_f = jnp.ones((L,), jnp.float32)

        pltpu.sync_copy(bi_hbm.at[pl.ds(sid * chunk, chunk)], bi_chunk)

        def scan_body(i, accs):
            v = bi_chunk[pl.ds(i * L, L)]
            return tuple(accs[k] + jnp.where(v < (k + 1), one_i, zero_i)
                         for k in range(B))

        accs = lax.fori_loop(0, n_vec, scan_body, (zero_i,) * B)
        for k in range(B):
            cnt_stage[k, :] = accs[k]
        pltpu.sync_copy(cnt_stage, cnt_sh.at[sid])
        plsc.subcore_barrier()
        pltpu.sync_copy(cnt_sh, cnt_all)

        totals = []
        for k in range(B):
            t = zero_i
            for w in range(NT):
                t = t + cnt_all[w, k, :]
            totals.append(_lane_total(t))

        def start_count_for(b):
            st_g = zero_i
            ct_g = zero_i
            prev = zero_i
            for k in range(B):
                sel = b == k
                st_g = jnp.where(sel, prev, st_g)
                ct_g = jnp.where(sel, totals[k] - prev, ct_g)
                prev = totals[k]
            return st_g, ct_g

        pbase = sid * per_tile
        pltpu.sync_copy(ind_hbm.at[pl.ds(pbase, per_tile)], ind_v)
        pltpu.sync_copy(mask_hbm.at[pl.ds(pbase, per_tile)], mask_v)
        pltpu.sync_copy(tgtT_hbm.at[:, pl.ds(pbase, per_tile)], tgt_v)

        numv = zero_f
        for g in range(groups):
            p = pbase + g * L + lane
            b = p // M
            st_g, ct_g = start_count_for(b)
            ind_g = ind_v[pl.ds(g * L, L)]
            clipped = jnp.clip(ind_g, 0, jnp.maximum(ct_g - one_i, zero_i))
            off = (st_g + clipped) * D
            blk0 = off // 128
            off_v[pl.ds(g * L, L)] = off
            b0_v[pl.ds(g * L, L)] = blk0
            b1_v[pl.ds(g * L, L)] = jnp.minimum(blk0 + one_i, nblk1)
            numv = numv + mask_v[pl.ds(g * L, L)].astype(jnp.float32)

        c0 = pltpu.async_copy(flat_hbm.at[b0_v], blk0_v, sem)
        c1 = pltpu.async_copy(flat_hbm.at[b1_v], blk1_v, sem)
        c0.wait()
        c1.wait()

        acc = [zero_f for _ in range(D)]
        for g in range(groups):
            p = pbase + g * L + lane
            b = p // M
            _, ct_g = start_count_for(b)
            cpos = ct_g > 0
            m = mask_v[pl.ds(g * L, L)].astype(jnp.float32)
            row_ids = jnp.full((L,), g * L, jnp.int32) + lane
            off_g = off_v[pl.ds(g * L, L)]
            blk0_g = off_g // 128
            for d in range(D):
                od = off_g + d
                woff = od - (od // 128) * 128
                g0 = plsc.load_gather(blk0_v, [row_ids, woff])
                g1 = plsc.load_gather(blk1_v, [row_ids, woff])
                pred = jnp.where(od // 128 == blk0_g, g0, g1)
                tgt = tgt_v[d, pl.ds(g * L, L)]
                pred = jnp.where(cpos, pred, zero_f)
                tn = tgt != tgt
                pn = pred != pred
                vm = m * jnp.where(tn, zero_f, one_f) \
                       * jnp.where(pn, zero_f, one_f)
                ps = jnp.where(pn, zero_f, pred)
                ts = jnp.where(tn, zero_f, tgt)
                acc[d] = acc[d] + jnp.abs(ps * vm - ts * vm)

        for d in range(D):
            part_stage[d, :] = acc[d]
        part_stage[D, :] = numv
        pltpu.sync_copy(part_stage, part_sh.at[sid])
        plsc.subcore_barrier()

        @pl.when(sid == 0)
        def _():
            pltpu.sync_copy(part_sh, part_all)
            loss = zero_f
            nv = zero_f
            for w in range(NT):
                nv = nv + part_all[w, D, :]
            for d in range(D):
                t = zero_f
                for w in range(NT):
                    t = t + part_all[w, d, :]
                loss = jnp.where(lane == d, _lane_total(t), loss)
            num = _lane_total(nv)
            loss = loss / jnp.maximum(num, one_f)
            loss = jnp.where(loss != loss, zero_f, loss)
            loss = jnp.where(num == 0.0, zero_f, loss)
            loss = jnp.where(lane < D, loss, zero_f)
            out_stage[...] = loss
            pltpu.sync_copy(out_stage, out_hbm)

    return sc_kernel


def kernel(output, mask, ind, target, batch_index):
    V, D = output.shape
    B, M = mask.shape
    N = batch_index.shape[0]
    P = B * M

    chunk = ((N + NT - 1) // NT + L - 1) // L * L
    N_pad = chunk * NT
    per_tile = ((P + NT - 1) // NT + L - 1) // L * L
    PP = per_tile * NT

    bi = jnp.pad(batch_index.astype(jnp.int32), (0, N_pad - N),
                 constant_values=B)
    ind_flat = jnp.pad(ind.reshape(-1).astype(jnp.int32), (0, PP - P))
    mask_flat = jnp.pad(mask.reshape(-1).astype(jnp.int32), (0, PP - P))
    tgtT = jnp.pad(target.reshape(P, D).astype(jnp.float32),
                   ((0, PP - P), (0, 0))).T

    NBLK = V * D // 128
    sc = _make_sc_call(chunk, per_tile, B, M, D, NBLK)
    out16 = sc(output.astype(jnp.float32).reshape(NBLK, 128),
               bi, ind_flat, mask_flat, tgtT)
    return out16[:D]


# speedup vs baseline: 6.4532x; 4.9756x over previous
import functools

import jax
import jax.numpy as jnp
from jax import lax
from jax.experimental import pallas as pl
from jax.experimental.pallas import tpu as pltpu
from jax.experimental.pallas import tpu_sc as plsc

L = 16
NT = 16


def _lane_total(v):
    return plsc.cummax(jnp.flip(plsc.cumsum(v), 0))


def _make_sc_call(chunk, per_tile, B, M, D, NV):
    n_vec = chunk // L
    groups = per_tile // L
    mesh = plsc.VectorSubcoreMesh(core_axis_name="c", subcore_axis_name="s",
                                  num_cores=1)

    @functools.partial(
        pl.kernel,
        out_type=jax.ShapeDtypeStruct((L,), jnp.float32),
        mesh=mesh,
        compiler_params=pltpu.CompilerParams(needs_layout_passes=False,
                                             use_tc_tiling_on_sc=False),
        scratch_types=[
            pltpu.VMEM((chunk,), jnp.int32),
            pltpu.VMEM((B, L), jnp.int32),
            pltpu.VMEM_SHARED((NT, B, L), jnp.int32),
            pltpu.VMEM((NT, B, L), jnp.int32),
            pltpu.VMEM((per_tile,), jnp.int32),
            pltpu.VMEM((per_tile,), jnp.int32),
            pltpu.VMEM((D, per_tile), jnp.int32),
            pltpu.VMEM((D, per_tile), jnp.float32),
            pltpu.VMEM((D, per_tile), jnp.float32),
            pltpu.VMEM((D + 1, L), jnp.float32),
            pltpu.VMEM_SHARED((NT, D + 1, L), jnp.float32),
            pltpu.VMEM((NT, D + 1, L), jnp.float32),
            pltpu.VMEM((L,), jnp.float32),
            pltpu.SemaphoreType.DMA,
        ],
    )
    def sc_kernel(flat_hbm, bi_hbm, ind_hbm, mask_hbm, tgtT_hbm, out_hbm,
                  bi_chunk, cnt_stage, cnt_sh, cnt_all, ind_v, mask_v,
                  idxd_v, cols_v, tgt_v, part_stage, part_sh, part_all,
                  out_stage, sem):
        sid = lax.axis_index("s")
        lane = jnp.arange(L, dtype=jnp.int32)
        zero_i = jnp.zeros((L,), jnp.int32)
        zero_f = jnp.zeros((L,), jnp.float32)
        one_i = jnp.ones((L,), jnp.int32)
        one_f = jnp.ones((L,), jnp.float32)

        pltpu.sync_copy(bi_hbm.at[pl.ds(sid * chunk, chunk)], bi_chunk)

        def scan_body(i, accs):
            v = bi_chunk[pl.ds(i * L, L)]
            return tuple(accs[k] + jnp.where(v < (k + 1), one_i, zero_i)
                         for k in range(B))

        accs = lax.fori_loop(0, n_vec, scan_body, (zero_i,) * B)
        for k in range(B):
            cnt_stage[k, :] = accs[k]
        pltpu.sync_copy(cnt_stage, cnt_sh.at[sid])
        plsc.subcore_barrier()
        pltpu.sync_copy(cnt_sh, cnt_all)

        totals = []
        for k in range(B):
            t = zero_i
            for w in range(NT):
                t = t + cnt_all[w, k, :]
            totals.append(_lane_total(t))

        def start_count_for(b):
            st_g = zero_i
            ct_g = zero_i
            prev = zero_i
            for k in range(B):
                sel = b == k
                st_g = jnp.where(sel, prev, st_g)
                ct_g = jnp.where(sel, totals[k] - prev, ct_g)
                prev = totals[k]
            return st_g, ct_g

        pbase = sid * per_tile
        pltpu.sync_copy(ind_hbm.at[pl.ds(pbase, per_tile)], ind_v)
        pltpu.sync_copy(mask_hbm.at[pl.ds(pbase, per_tile)], mask_v)
        pltpu.sync_copy(tgtT_hbm.at[:, pl.ds(pbase, per_tile)], tgt_v)

        numv = zero_f
        for g in range(groups):
            p = pbase + g * L + lane
            b = p // M
            st_g, ct_g = start_count_for(b)
            ind_g = ind_v[pl.ds(g * L, L)]
            clipped = jnp.clip(ind_g, 0, jnp.maximum(ct_g - one_i, zero_i))
            row = st_g + clipped
            for d in range(D):
                idxd_v[d, pl.ds(g * L, L)] = row + d * NV
            numv = numv + mask_v[pl.ds(g * L, L)].astype(jnp.float32)

        copies = [pltpu.async_copy(flat_hbm.at[idxd_v.at[d]], cols_v.at[d],
                                   sem) for d in range(D)]
        for c in copies:
            c.wait()

        acc = [zero_f for _ in range(D)]
        for g in range(groups):
            p = pbase + g * L + lane
            b = p // M
            _, ct_g = start_count_for(b)
            cpos = ct_g > 0
            m = mask_v[pl.ds(g * L, L)].astype(jnp.float32)
            for d in range(D):
                pred = cols_v[d, pl.ds(g * L, L)]
                tgt = tgt_v[d, pl.ds(g * L, L)]
                pred = jnp.where(cpos, pred, zero_f)
                tn = tgt != tgt
                pn = pred != pred
                vm = m * jnp.where(tn, zero_f, one_f) \
                       * jnp.where(pn, zero_f, one_f)
                ps = jnp.where(pn, zero_f, pred)
                ts = jnp.where(tn, zero_f, tgt)
                acc[d] = acc[d] + jnp.abs(ps * vm - ts * vm)

        for d in range(D):
            part_stage[d, :] = acc[d]
        part_stage[D, :] = numv
        pltpu.sync_copy(part_stage, part_sh.at[sid])
        plsc.subcore_barrier()

        @pl.when(sid == 0)
        def _():
            pltpu.sync_copy(part_sh, part_all)
            loss = zero_f
            nv = zero_f
            for w in range(NT):
                nv = nv + part_all[w, D, :]
            for d in range(D):
                t = zero_f
                for w in range(NT):
                    t = t + part_all[w, d, :]
                loss = jnp.where(lane == d, _lane_total(t), loss)
            num = _lane_total(nv)
            loss = loss / jnp.maximum(num, one_f)
            loss = jnp.where(loss != loss, zero_f, loss)
            loss = jnp.where(num == 0.0, zero_f, loss)
            loss = jnp.where(lane < D, loss, zero_f)
            out_stage[...] = loss
            pltpu.sync_copy(out_stage, out_hbm)

    return sc_kernel


def kernel(output, mask, ind, target, batch_index):
    V, D = output.shape
    B, M = mask.shape
    N = batch_index.shape[0]
    P = B * M

    chunk = ((N + NT - 1) // NT + L - 1) // L * L
    N_pad = chunk * NT
    per_tile = ((P + NT - 1) // NT + L - 1) // L * L
    PP = per_tile * NT

    bi = jnp.pad(batch_index.astype(jnp.int32), (0, N_pad - N),
                 constant_values=B)
    ind_flat = jnp.pad(ind.reshape(-1).astype(jnp.int32), (0, PP - P))
    mask_flat = jnp.pad(mask.reshape(-1).astype(jnp.int32), (0, PP - P))
    tgtT = jnp.pad(target.reshape(P, D).astype(jnp.float32),
                   ((0, PP - P), (0, 0))).T

    sc = _make_sc_call(chunk, per_tile, B, M, D, V)
    out16 = sc(output.astype(jnp.float32).T.reshape(-1),
               bi, ind_flat, mask_flat, tgtT)
    return out16[:D]


# speedup vs baseline: 6.4722x; 1.0030x over previous
import functools

import jax
import jax.numpy as jnp
from jax import lax
from jax.experimental import pallas as pl
from jax.experimental.pallas import tpu as pltpu
from jax.experimental.pallas import tpu_sc as plsc

L = 16
NT = 16


def _lane_total(v):
    return plsc.cummax(jnp.flip(plsc.cumsum(v), 0))


def _make_sc_call(chunk, per_tile, B, M, D, NV):
    n_vec = chunk // L
    groups = per_tile // L
    UNROLL = 4 if n_vec % 4 == 0 else 1
    mesh = plsc.VectorSubcoreMesh(core_axis_name="c", subcore_axis_name="s",
                                  num_cores=1)

    @functools.partial(
        pl.kernel,
        out_type=jax.ShapeDtypeStruct((L,), jnp.float32),
        mesh=mesh,
        compiler_params=pltpu.CompilerParams(needs_layout_passes=False,
                                             use_tc_tiling_on_sc=False),
        scratch_types=[
            pltpu.VMEM((chunk,), jnp.int32),
            pltpu.VMEM((B, L), jnp.int32),
            pltpu.VMEM_SHARED((NT, B, L), jnp.int32),
            pltpu.VMEM((NT, B, L), jnp.int32),
            pltpu.VMEM((per_tile,), jnp.int32),
            pltpu.VMEM((per_tile,), jnp.int32),
            pltpu.VMEM((D, per_tile), jnp.int32),
            pltpu.VMEM((D, per_tile), jnp.float32),
            pltpu.VMEM((D, per_tile), jnp.float32),
            pltpu.VMEM((D + 1, L), jnp.float32),
            pltpu.VMEM_SHARED((NT, D + 1, L), jnp.float32),
            pltpu.VMEM((NT, D + 1, L), jnp.float32),
            pltpu.VMEM((L,), jnp.float32),
            pltpu.SemaphoreType.DMA,
        ],
    )
    def sc_kernel(flat_hbm, bi_hbm, ind_hbm, mask_hbm, tgtT_hbm, out_hbm,
                  bi_chunk, cnt_stage, cnt_sh, cnt_all, ind_v, mask_v,
                  idxd_v, cols_v, tgt_v, part_stage, part_sh, part_all,
                  out_stage, sem):
        sid = lax.axis_index("s")
        lane = jnp.arange(L, dtype=jnp.int32)
        zero_i = jnp.zeros((L,), jnp.int32)
        zero_f = jnp.zeros((L,), jnp.float32)
        one_i = jnp.ones((L,), jnp.int32)
        one_f = jnp.ones((L,), jnp.float32)

        pltpu.sync_copy(bi_hbm.at[pl.ds(sid * chunk, chunk)], bi_chunk)

        def scan_body(i, accs):
            accs = list(accs)
            for u in range(UNROLL):
                v = bi_chunk[pl.ds((i * UNROLL + u) * L, L)]
                for k in range(B):
                    accs[k] = accs[k] + jnp.minimum(v, k + 1)
            return tuple(accs)

        accs = lax.fori_loop(0, n_vec // UNROLL, scan_body, (zero_i,) * B)
        for k in range(B):
            cnt_stage[k, :] = accs[k]
        pltpu.sync_copy(cnt_stage, cnt_sh.at[sid])
        plsc.subcore_barrier()
        pltpu.sync_copy(cnt_sh, cnt_all)

        npad_v = jnp.full((L,), chunk * NT, jnp.int32)
        totals = []
        prev_a = zero_i
        for k in range(B):
            t = zero_i
            for w in range(NT):
                t = t + cnt_all[w, k, :]
            a_k = _lane_total(t)
            totals.append(npad_v - (a_k - prev_a))
            prev_a = a_k

        def start_count_for(b):
            st_g = zero_i
            ct_g = zero_i
            prev = zero_i
            for k in range(B):
                sel = b == k
                st_g = jnp.where(sel, prev, st_g)
                ct_g = jnp.where(sel, totals[k] - prev, ct_g)
                prev = totals[k]
            return st_g, ct_g

        pbase = sid * per_tile
        pltpu.sync_copy(ind_hbm.at[pl.ds(pbase, per_tile)], ind_v)
        pltpu.sync_copy(mask_hbm.at[pl.ds(pbase, per_tile)], mask_v)
        pltpu.sync_copy(tgtT_hbm.at[:, pl.ds(pbase, per_tile)], tgt_v)

        numv = zero_f
        for g in range(groups):
            p = pbase + g * L + lane
            b = p // M
            st_g, ct_g = start_count_for(b)
            ind_g = ind_v[pl.ds(g * L, L)]
            clipped = jnp.clip(ind_g, 0, jnp.maximum(ct_g - one_i, zero_i))
            row = st_g + clipped
            for d in range(D):
                idxd_v[d, pl.ds(g * L, L)] = row + d * NV
            numv = numv + mask_v[pl.ds(g * L, L)].astype(jnp.float32)

        copies = [pltpu.async_copy(flat_hbm.at[idxd_v.at[d]], cols_v.at[d],
                                   sem) for d in range(D)]
        for c in copies:
            c.wait()

        acc = [zero_f for _ in range(D)]
        for g in range(groups):
            p = pbase + g * L + lane
            b = p // M
            _, ct_g = start_count_for(b)
            cpos = ct_g > 0
            m = mask_v[pl.ds(g * L, L)].astype(jnp.float32)
            for d in range(D):
                pred = cols_v[d, pl.ds(g * L, L)]
                tgt = tgt_v[d, pl.ds(g * L, L)]
                pred = jnp.where(cpos, pred, zero_f)
                tn = tgt != tgt
                pn = pred != pred
                vm = m * jnp.where(tn, zero_f, one_f) \
                       * jnp.where(pn, zero_f, one_f)
                ps = jnp.where(pn, zero_f, pred)
                ts = jnp.where(tn, zero_f, tgt)
                acc[d] = acc[d] + jnp.abs(ps * vm - ts * vm)

        for d in range(D):
            part_stage[d, :] = acc[d]
        part_stage[D, :] = numv
        pltpu.sync_copy(part_stage, part_sh.at[sid])
        plsc.subcore_barrier()

        @pl.when(sid == 0)
        def _():
            pltpu.sync_copy(part_sh, part_all)
            loss = zero_f
            nv = zero_f
            for w in range(NT):
                nv = nv + part_all[w, D, :]
            for d in range(D):
                t = zero_f
                for w in range(NT):
                    t = t + part_all[w, d, :]
                loss = jnp.where(lane == d, _lane_total(t), loss)
            num = _lane_total(nv)
            loss = loss / jnp.maximum(num, one_f)
            loss = jnp.where(loss != loss, zero_f, loss)
            loss = jnp.where(num == 0.0, zero_f, loss)
            loss = jnp.where(lane < D, loss, zero_f)
            out_stage[...] = loss
            pltpu.sync_copy(out_stage, out_hbm)

    return sc_kernel


def kernel(output, mask, ind, target, batch_index):
    V, D = output.shape
    B, M = mask.shape
    N = batch_index.shape[0]
    P = B * M

    chunk = ((N + NT - 1) // NT + L - 1) // L * L
    N_pad = chunk * NT
    per_tile = ((P + NT - 1) // NT + L - 1) // L * L
    PP = per_tile * NT

    bi = jnp.pad(batch_index.astype(jnp.int32), (0, N_pad - N),
                 constant_values=B)
    ind_flat = jnp.pad(ind.reshape(-1).astype(jnp.int32), (0, PP - P))
    mask_flat = jnp.pad(mask.reshape(-1).astype(jnp.int32), (0, PP - P))
    tgtT = jnp.pad(target.reshape(P, D).astype(jnp.float32),
                   ((0, PP - P), (0, 0))).T

    sc = _make_sc_call(chunk, per_tile, B, M, D, V)
    out16 = sc(output.astype(jnp.float32).T.reshape(-1),
               bi, ind_flat, mask_flat, tgtT)
    return out16[:D]
